# baseline (speedup 1.0000x reference)
import functools

import jax
import jax.numpy as jnp
from jax import lax
from jax.experimental import pallas as pl
from jax.experimental.pallas import tpu as pltpu
from jax.experimental.pallas import tpu_sc as plsc

NUM_FRAMES = 16
VIDEO_VOCABS = 8192
MASK_TOKEN = VIDEO_VOCABS
B = 64
FRAME_L = 1024
TOTAL_L = NUM_FRAMES * FRAME_L
NW = 32


def _rotl(x, d):
    return lax.shift_left(x, jnp.int32(d)) | lax.shift_right_logical(
        x, jnp.int32(32 - d)
    )


def _threefry2x32(ks0, ks1, x0, x1):
    ks2 = ks0 ^ ks1 ^ jnp.int32(0x1BD11BDA)
    ks = [ks0, ks1, ks2]
    rots = ((13, 15, 26, 6), (17, 29, 16, 24))
    x0 = x0 + ks0
    x1 = x1 + ks1
    for i in range(5):
        for r in rots[i % 2]:
            x0 = x0 + x1
            x1 = _rotl(x1, r)
            x1 = x0 ^ x1
        x0 = x0 + ks[(i + 1) % 3]
        x1 = x1 + ks[(i + 2) % 3] + jnp.int32(i + 1)
    return x0, x1


def _keep_kernel(keep_ref):
    frame = pl.program_id(0)
    k0, k1 = _threefry2x32(jnp.int32(0), jnp.int32(42), jnp.int32(0), frame)
    row = lax.broadcasted_iota(jnp.int32, (B, FRAME_L), 0)
    col = lax.broadcasted_iota(jnp.int32, (B, FRAME_L), 1)
    cnt = row * FRAME_L + col
    o0, o1 = _threefry2x32(k0, k1, jnp.zeros((B, FRAME_L), jnp.int32), cnt)
    ki = lax.shift_right_logical(o0 ^ o1, 9)
    m = jnp.max(ki, axis=1, keepdims=True)
    keep = jnp.max(jnp.where(ki == m, col, -1), axis=1, keepdims=True)
    keep_ref[...] = jnp.broadcast_to(keep, (B, 128))


def _keep_indices(mo_tokens):
    del mo_tokens
    padded = pl.pallas_call(
        _keep_kernel,
        grid=(NUM_FRAMES,),
        in_specs=[],
        out_specs=pl.BlockSpec((B, 128), lambda i: (0, i)),
        out_shape=jax.ShapeDtypeStruct((B, NUM_FRAMES * 128), jnp.int32),
    )()
    return padded[:, ::128]


N128 = B * TOTAL_L // 128


def _sc_scatter_kernel(keep_hbm, tok128_hbm, out128_hbm, msk128_hbm,
                       keep_v, idx_v, chunks_v, fixtok_v, fixmsk_v,
                       mfill_v, onefill_v, sem_fill, sem_g):
    wid = lax.axis_index("s") * 2 + lax.axis_index("c")
    b0 = wid * 2
    mask16 = jnp.full((16,), MASK_TOKEN, jnp.int32)
    one16 = jnp.ones((16,), jnp.int32)
    for k in range(8):
        for h in range(8):
            mfill_v[k, pl.ds(h * 16, 16)] = mask16
            onefill_v[k, pl.ds(h * 16, 16)] = one16

    fills = []
    for r in range(2):
        for t in range(NUM_FRAMES):
            seg = (b0 + r) * 128 + t * 8
            fills.append(pltpu.async_copy(
                mfill_v, out128_hbm.at[pl.ds(seg, 8)], sem_fill))
            fills.append(pltpu.async_copy(
                onefill_v, msk128_hbm.at[pl.ds(seg, 8)], sem_fill))

    pltpu.sync_copy(keep_hbm.at[pl.ds(b0 * NUM_FRAMES, 32)], keep_v)

    t16 = lax.iota(jnp.int32, 16)
    kv = []
    for r in range(2):
        kvr = keep_v[pl.ds(r * 16, 16)]
        kv.append(kvr)
        idx_v[pl.ds(r * 16, 16)] = (
            (b0 + r) * 128 + t16 * 8 + lax.shift_right_logical(kvr, 7)
        )
    pltpu.async_copy(tok128_hbm.at[idx_v], chunks_v, sem_g).wait()

    for j in range(32):
        koff = kv[j // 16][j % 16] & 127
        for h in range(8):
            sel = (t16 + h * 16) == koff
            fixtok_v[j, pl.ds(h * 16, 16)] = jnp.where(
                sel, chunks_v[j, pl.ds(h * 16, 16)], MASK_TOKEN)
            fixmsk_v[j, pl.ds(h * 16, 16)] = jnp.where(sel, 0, 1)

    for f in fills:
        f.wait()

    w1 = pltpu.async_copy(fixtok_v, out128_hbm.at[idx_v], sem_g)
    w2 = pltpu.async_copy(fixmsk_v, msk128_hbm.at[idx_v], sem_g)
    w1.wait()
    w2.wait()


@functools.partial(
    pl.kernel,
    mesh=plsc.VectorSubcoreMesh(core_axis_name="c", subcore_axis_name="s"),
    out_type=[
        jax.ShapeDtypeStruct((N128, 128), jnp.int32),
        jax.ShapeDtypeStruct((N128, 128), jnp.int32),
    ],
    scratch_types=[
        pltpu.VMEM((32,), jnp.int32),
        pltpu.VMEM((32,), jnp.int32),
        pltpu.VMEM((32, 128), jnp.int32),
        pltpu.VMEM((32, 128), jnp.int32),
        pltpu.VMEM((32, 128), jnp.int32),
        pltpu.VMEM((8, 128), jnp.int32),
        pltpu.VMEM((8, 128), jnp.int32),
        pltpu.SemaphoreType.DMA,
        pltpu.SemaphoreType.DMA,
    ],
)
def _sc_scatter(keep_flat, tok128, *rest):
    _sc_scatter_kernel(keep_flat, tok128, *rest)


def kernel(mo_tokens, rate):
    del rate
    keep = _keep_indices(mo_tokens)
    keep_flat = keep.reshape(B * NUM_FRAMES)
    tok128 = mo_tokens.reshape(N128, 128)
    out128, msk128 = _sc_scatter(keep_flat, tok128)
    return out128.reshape(B, TOTAL_L), msk128.reshape(B, TOTAL_L)


# speedup vs baseline: 1.0076x; 1.0076x over previous
import functools

import jax
import jax.numpy as jnp
from jax import lax
from jax.experimental import pallas as pl
from jax.experimental.pallas import tpu as pltpu
from jax.experimental.pallas import tpu_sc as plsc

NUM_FRAMES = 16
VIDEO_VOCABS = 8192
MASK_TOKEN = VIDEO_VOCABS
B = 64
FRAME_L = 1024
TOTAL_L = NUM_FRAMES * FRAME_L
NW = 32


def _rotl(x, d):
    return lax.shift_left(x, jnp.int32(d)) | lax.shift_right_logical(
        x, jnp.int32(32 - d)
    )


def _threefry2x32(ks0, ks1, x0, x1):
    ks2 = ks0 ^ ks1 ^ jnp.int32(0x1BD11BDA)
    ks = [ks0, ks1, ks2]
    rots = ((13, 15, 26, 6), (17, 29, 16, 24))
    x0 = x0 + ks0
    x1 = x1 + ks1
    for i in range(5):
        for r in rots[i % 2]:
            x0 = x0 + x1
            x1 = _rotl(x1, r)
            x1 = x0 ^ x1
        x0 = x0 + ks[(i + 1) % 3]
        x1 = x1 + ks[(i + 2) % 3] + jnp.int32(i + 1)
    return x0, x1


def _keep_kernel(keep_ref):
    frame = pl.program_id(0)
    k0, k1 = _threefry2x32(jnp.int32(0), jnp.int32(42), jnp.int32(0), frame)
    row = lax.broadcasted_iota(jnp.int32, (B, FRAME_L), 0)
    col = lax.broadcasted_iota(jnp.int32, (B, FRAME_L), 1)
    cnt = row * FRAME_L + col
    o0, o1 = _threefry2x32(k0, k1, jnp.zeros((B, FRAME_L), jnp.int32), cnt)
    ki = lax.shift_right_logical(o0 ^ o1, 9)
    m = jnp.max(ki, axis=1, keepdims=True)
    keep = jnp.max(jnp.where(ki == m, col, -1), axis=1, keepdims=True)
    keep_ref[...] = jnp.broadcast_to(keep, (B, 128))


def _keep_indices(mo_tokens):
    del mo_tokens
    return pl.pallas_call(
        _keep_kernel,
        grid=(NUM_FRAMES,),
        in_specs=[],
        out_specs=pl.BlockSpec((B, 128), lambda i: (0, i)),
        out_shape=jax.ShapeDtypeStruct((B, NUM_FRAMES * 128), jnp.int32),
    )()


N128 = B * TOTAL_L // 128


def _sc_scatter_kernel(keep_hbm, tok128_hbm, out128_hbm, msk128_hbm,
                       keep_v, idx_v, chunks_v, fixtok_v, fixmsk_v,
                       mfill_v, onefill_v, sem_fill, sem_g, sem_k):
    wid = lax.axis_index("s") * 2 + lax.axis_index("c")
    b0 = wid * 2
    kcopy = pltpu.async_copy(
        keep_hbm.at[pl.ds(b0 * NUM_FRAMES, 32)], keep_v, sem_k)

    mask16 = jnp.full((16,), MASK_TOKEN, jnp.int32)
    one16 = jnp.ones((16,), jnp.int32)
    for k in range(32):
        for h in range(8):
            mfill_v[k, pl.ds(h * 16, 16)] = mask16
            onefill_v[k, pl.ds(h * 16, 16)] = one16

    fills = []
    for r in range(2):
        for q in range(4):
            seg = (b0 + r) * 128 + q * 32
            fills.append(pltpu.async_copy(
                mfill_v, out128_hbm.at[pl.ds(seg, 32)], sem_fill))
            fills.append(pltpu.async_copy(
                onefill_v, msk128_hbm.at[pl.ds(seg, 32)], sem_fill))

    kcopy.wait()

    t16 = lax.iota(jnp.int32, 16)
    z16 = jnp.zeros((16,), jnp.int32)
    kv = []
    for r in range(2):
        kvr = z16
        for t in range(NUM_FRAMES):
            kvr = jnp.where(t16 == t, keep_v[r * 16 + t, pl.ds(0, 16)], kvr)
        kv.append(kvr)
        idx_v[pl.ds(r * 16, 16)] = (
            (b0 + r) * 128 + t16 * 8 + lax.shift_right_logical(kvr, 7)
        )
    pltpu.async_copy(tok128_hbm.at[idx_v], chunks_v, sem_g).wait()

    for j in range(32):
        koff = kv[j // 16][j % 16] & 127
        for h in range(8):
            sel = (t16 + h * 16) == koff
            fixtok_v[j, pl.ds(h * 16, 16)] = jnp.where(
                sel, chunks_v[j, pl.ds(h * 16, 16)], MASK_TOKEN)
            fixmsk_v[j, pl.ds(h * 16, 16)] = jnp.where(sel, 0, 1)

    for f in fills:
        f.wait()

    w1 = pltpu.async_copy(fixtok_v, out128_hbm.at[idx_v], sem_g)
    w2 = pltpu.async_copy(fixmsk_v, msk128_hbm.at[idx_v], sem_g)
    w1.wait()
    w2.wait()


@functools.partial(
    pl.kernel,
    mesh=plsc.VectorSubcoreMesh(core_axis_name="c", subcore_axis_name="s"),
    out_type=[
        jax.ShapeDtypeStruct((N128, 128), jnp.int32),
        jax.ShapeDtypeStruct((N128, 128), jnp.int32),
    ],
    scratch_types=[
        pltpu.VMEM((32, 128), jnp.int32),
        pltpu.VMEM((32,), jnp.int32),
        pltpu.VMEM((32, 128), jnp.int32),
        pltpu.VMEM((32, 128), jnp.int32),
        pltpu.VMEM((32, 128), jnp.int32),
        pltpu.VMEM((32, 128), jnp.int32),
        pltpu.VMEM((32, 128), jnp.int32),
        pltpu.SemaphoreType.DMA,
        pltpu.SemaphoreType.DMA,
        pltpu.SemaphoreType.DMA,
    ],
)
def _sc_scatter(keep_pad, tok128, *rest):
    _sc_scatter_kernel(keep_pad, tok128, *rest)


def kernel(mo_tokens, rate):
    del rate
    keep_pad = _keep_indices(mo_tokens).reshape(B * NUM_FRAMES, 128)
    tok128 = mo_tokens.reshape(N128, 128)
    out128, msk128 = _sc_scatter(keep_pad, tok128)
    return out128.reshape(B, TOTAL_L), msk128.reshape(B, TOTAL_L)
